# baseline (speedup 1.0000x reference)
import functools

import jax
import jax.numpy as jnp
from jax import lax
from jax.experimental import pallas as pl
from jax.experimental.pallas import tpu as pltpu
from jax.experimental.pallas import tpu_sc as plsc

D = 32
L = 16
NCORE = 2
NSUB = 16
NW = NCORE * NSUB
NROW = 4096
NCOL = 200
TI = NROW // 128


@functools.partial(
    pl.kernel,
    mesh=plsc.VectorSubcoreMesh(core_axis_name="c", subcore_axis_name="s"),
    out_type=jax.ShapeDtypeStruct((NCOL, D // 8, TI, 8, 128), jnp.float32),
    compiler_params=pltpu.CompilerParams(
        needs_layout_passes=False, use_tc_tiling_on_sc=False),
    scratch_types=[
        pltpu.VMEM((128,), jnp.int32),
        pltpu.VMEM((128,), jnp.int32),
        pltpu.VMEM((128, D), jnp.float32),
        pltpu.VMEM((128, D), jnp.float32),
        pltpu.VMEM((D // 8, 8, 128), jnp.float32),
        pltpu.VMEM((D // 8, 8, 128), jnp.float32),
        pltpu.SemaphoreType.DMA,
        pltpu.SemaphoreType.DMA,
        pltpu.SemaphoreType.DMA,
        pltpu.SemaphoreType.DMA,
        pltpu.SemaphoreType.DMA,
        pltpu.SemaphoreType.DMA,
    ],
)
def _sc_embedding(x_hbm, table_hbm, out_hbm,
                  idx0, idx1, rows0, rows1, t0, t1,
                  si0, si1, sg0, sg1, ss0, ss1):
    idxb = (idx0, idx1)
    rowsb = (rows0, rows1)
    tb = (t0, t1)
    sidx = (si0, si1)
    sgat = (sg0, sg1)
    ssto = (ss0, ss1)

    wid = lax.axis_index("s") * NCORE + lax.axis_index("c")

    def idx_copy(j, b):
        return pltpu.make_async_copy(x_hbm.at[j, wid], idxb[b], sidx[b])

    def gather_copy(b):
        return pltpu.make_async_copy(
            table_hbm.at[idxb[b]], rowsb[b], sgat[b])

    def gather_drain(b):
        pltpu.make_async_copy(
            table_hbm.at[pl.ds(0, 128)], rowsb[b], sgat[b]).wait()

    def store_copy(j, b):
        return pltpu.make_async_copy(
            tb[b], out_hbm.at[j, pl.ds(0, D // 8), wid], ssto[b])

    def transpose_mask(b):
        rows = rowsb[b]
        out = tb[b]
        idxr = idxb[b]

        def grp(g, carry):
            li0 = g * L
            vi = idxr[pl.ds(li0, L)]
            m = jnp.where(vi == 0, 0.0, 1.0)
            rids = lax.iota(jnp.int32, L) + li0
            for k in range(D):
                v = plsc.load_gather(
                    rows, [rids, jnp.full((L,), k, jnp.int32)])
                out[k // 8, k % 8, pl.ds(li0, L)] = v * m
            return carry

        lax.fori_loop(0, 128 // L, grp, 0)

    def process(j, b):
        ob = 1 - b

        @pl.when(j + 1 < NCOL)
        def _():
            idx_copy(j + 1, ob).wait()

            @pl.when(j >= 1)
            def _():
                store_copy(j - 1, ob).wait()

            gather_copy(ob).start()

        gather_drain(b)
        transpose_mask(b)

        @pl.when(j + 2 < NCOL)
        def _():
            idx_copy(j + 2, b).start()

        store_copy(j, b).start()

    idx_copy(0, 0).start()
    idx_copy(1, 1).start()
    idx_copy(0, 0).wait()
    gather_copy(0).start()

    def outer(J, carry):
        process(2 * J, 0)
        process(2 * J + 1, 1)
        return carry

    lax.fori_loop(0, NCOL // 2, outer, 0)

    store_copy(NCOL - 2, 0).wait()
    store_copy(NCOL - 1, 1).wait()


def kernel(x, table):
    assert x.shape == (NROW, NCOL) and table.shape[1] == D
    x5 = x.T.reshape(NCOL, TI, 128)
    out5 = _sc_embedding(x5, table)
    return out5.transpose(2, 4, 0, 1, 3).reshape(NROW, NCOL, D)


# speedup vs baseline: 1.3437x; 1.3437x over previous
import functools

import jax
import jax.numpy as jnp
from jax import lax
from jax.experimental import pallas as pl
from jax.experimental.pallas import tpu as pltpu
from jax.experimental.pallas import tpu_sc as plsc

D = 32
L = 16
NCORE = 2
NSUB = 16
NW = NCORE * NSUB
NROW = 4096
NCOL = 200
TI = NROW // 128
JB = 4
NB = NCOL // JB


@functools.partial(
    pl.kernel,
    mesh=plsc.VectorSubcoreMesh(core_axis_name="c", subcore_axis_name="s"),
    out_type=jax.ShapeDtypeStruct((NCOL, D // 8, TI, 8, 128), jnp.float32),
    compiler_params=pltpu.CompilerParams(
        needs_layout_passes=False, use_tc_tiling_on_sc=False),
    scratch_types=[
        pltpu.VMEM((JB, 128), jnp.int32),
        pltpu.VMEM((JB, 128), jnp.int32),
        pltpu.VMEM((JB, 128, D), jnp.float32),
        pltpu.VMEM((JB, 128, D), jnp.float32),
        pltpu.VMEM((JB, D // 8, 8, 128), jnp.float32),
        pltpu.VMEM((JB, D // 8, 8, 128), jnp.float32),
        pltpu.SemaphoreType.DMA,
        pltpu.SemaphoreType.DMA,
        pltpu.SemaphoreType.DMA,
        pltpu.SemaphoreType.DMA,
        pltpu.SemaphoreType.DMA,
        pltpu.SemaphoreType.DMA,
    ],
)
def _sc_embedding(x_hbm, table_hbm, out_hbm,
                  idx0, idx1, rows0, rows1, t0, t1,
                  si0, si1, sg0, sg1, ss0, ss1):
    idxb = (idx0, idx1)
    rowsb = (rows0, rows1)
    tb = (t0, t1)
    sidx = (si0, si1)
    sgat = (sg0, sg1)
    ssto = (ss0, ss1)

    wid = lax.axis_index("s") * NCORE + lax.axis_index("c")

    def idx_copy(n, b):
        return pltpu.make_async_copy(
            x_hbm.at[pl.ds(n * JB, JB), wid], idxb[b], sidx[b])

    def gather_start(b):
        for q in range(JB):
            pltpu.make_async_copy(
                table_hbm.at[idxb[b].at[q]], rowsb[b].at[q], sgat[b]).start()

    def gather_drain(b):
        for q in range(JB):
            pltpu.make_async_copy(
                table_hbm.at[pl.ds(0, 128)], rowsb[b].at[q], sgat[b]).wait()

    def store_copy(n, b):
        return pltpu.make_async_copy(
            tb[b],
            out_hbm.at[pl.ds(n * JB, JB), pl.ds(0, D // 8), wid],
            ssto[b])

    def transpose_mask(b):
        rows = rowsb[b]
        out = tb[b]
        idxr = idxb[b]

        @plsc.parallel_loop(0, JB * (128 // L), unroll=2)
        def _(i):
            q = i // (128 // L)
            li0 = (i % (128 // L)) * L
            vi = idxr[q, pl.ds(li0, L)]
            m = jnp.where(vi == 0, 0.0, 1.0)
            rids = lax.iota(jnp.int32, L) + li0
            qids = jnp.full((L,), q, jnp.int32)
            for k in range(D):
                v = plsc.load_gather(
                    rows, [qids, rids, jnp.full((L,), k, jnp.int32)])
                out[q, k // 8, k % 8, pl.ds(li0, L)] = v * m

    def process(n, b):
        ob = 1 - b

        @pl.when(n + 1 < NB)
        def _():
            idx_copy(n + 1, ob).wait()

            @pl.when(n >= 1)
            def _():
                store_copy(n - 1, ob).wait()

            gather_start(ob)

        gather_drain(b)
        transpose_mask(b)

        @pl.when(n + 2 < NB)
        def _():
            idx_copy(n + 2, b).start()

        store_copy(n, b).start()

    idx_copy(0, 0).start()
    idx_copy(1, 1).start()
    idx_copy(0, 0).wait()
    gather_start(0)

    def outer(G, carry):
        process(2 * G, 0)
        process(2 * G + 1, 1)
        return carry

    lax.fori_loop(0, NB // 2, outer, 0)

    store_copy(NB - 2, 0).wait()
    store_copy(NB - 1, 1).wait()


def kernel(x, table):
    assert x.shape == (NROW, NCOL) and table.shape[1] == D
    x5 = x.T.reshape(NCOL, TI, 128)
    out5 = _sc_embedding(x5, table)
    return out5.transpose(2, 4, 0, 1, 3).reshape(NROW, NCOL, D)


# speedup vs baseline: 1.3444x; 1.0005x over previous
import functools

import jax
import jax.numpy as jnp
from jax import lax
from jax.experimental import pallas as pl
from jax.experimental.pallas import tpu as pltpu
from jax.experimental.pallas import tpu_sc as plsc

D = 32
L = 16
NCORE = 2
NSUB = 16
NW = NCORE * NSUB
NROW = 4096
NCOL = 200
TI = NROW // 128
JB = 4
NB = NCOL // JB


@functools.partial(
    pl.kernel,
    mesh=plsc.VectorSubcoreMesh(core_axis_name="c", subcore_axis_name="s"),
    out_type=jax.ShapeDtypeStruct((NCOL, D // 8, TI, 8, 128), jnp.float32),
    compiler_params=pltpu.CompilerParams(
        needs_layout_passes=False, use_tc_tiling_on_sc=False),
    scratch_types=[
        pltpu.VMEM((JB, 128), jnp.int32),
        pltpu.VMEM((JB, 128), jnp.int32),
        pltpu.VMEM((JB, 128, D), jnp.float32),
        pltpu.VMEM((JB, 128, D), jnp.float32),
        pltpu.VMEM((JB, D // 8, 8, 128), jnp.float32),
        pltpu.VMEM((JB, D // 8, 8, 128), jnp.float32),
        pltpu.SemaphoreType.DMA,
        pltpu.SemaphoreType.DMA,
        pltpu.SemaphoreType.DMA,
        pltpu.SemaphoreType.DMA,
        pltpu.SemaphoreType.DMA,
        pltpu.SemaphoreType.DMA,
    ],
)
def _sc_embedding(x_hbm, table_hbm, out_hbm,
                  idx0, idx1, rows0, rows1, t0, t1,
                  si0, si1, sg0, sg1, ss0, ss1):
    idxb = (idx0, idx1)
    rowsb = (rows0, rows1)
    tb = (t0, t1)
    sidx = (si0, si1)
    sgat = (sg0, sg1)
    ssto = (ss0, ss1)

    wid = lax.axis_index("s") * NCORE + lax.axis_index("c")


    def idx_copy(n, b):
        u = wid + NW * n
        return pltpu.make_async_copy(
            x_hbm.at[u // 8, pl.ds((u % 8) * JB, JB)], idxb[b], sidx[b])

    def gather_start(b):
        for q in range(JB):
            pltpu.make_async_copy(
                table_hbm.at[idxb[b].at[q]], rowsb[b].at[q], sgat[b]).start()

    def gather_drain(b):
        for q in range(JB):
            pltpu.make_async_copy(
                table_hbm.at[pl.ds(0, 128)], rowsb[b].at[q], sgat[b]).wait()

    def store_copy(n, b):
        u = wid + NW * n
        return pltpu.make_async_copy(
            tb[b],
            out_hbm.at[u // 8, pl.ds(0, D // 8), pl.ds((u % 8) * JB, JB)],
            ssto[b])

    def transpose_mask(b):
        rows = rowsb[b]
        out = tb[b]
        idxr = idxb[b]

        @plsc.parallel_loop(0, JB * (128 // L), unroll=2)
        def _(i):
            q = i // (128 // L)
            li0 = (i % (128 // L)) * L
            vi = idxr[q, pl.ds(li0, L)]
            m = jnp.where(vi == 0, 0.0, 1.0)
            rids = lax.iota(jnp.int32, L) + li0
            qids = jnp.full((L,), q, jnp.int32)
            for k in range(D):
                v = plsc.load_gather(
                    rows, [qids, rids, jnp.full((L,), k, jnp.int32)])
                out[k // 8, q, k % 8, pl.ds(li0, L)] = v * m

    def process(n, b):
        ob = 1 - b

        @pl.when(n + 1 < NB)
        def _():
            idx_copy(n + 1, ob).wait()

            @pl.when(n >= 1)
            def _():
                store_copy(n - 1, ob).wait()

            gather_start(ob)

        gather_drain(b)
        transpose_mask(b)

        @pl.when(n + 2 < NB)
        def _():
            idx_copy(n + 2, b).start()

        store_copy(n, b).start()

    idx_copy(0, 0).start()
    idx_copy(1, 1).start()
    idx_copy(0, 0).wait()
    gather_start(0)

    def outer(G, carry):
        process(2 * G, 0)
        process(2 * G + 1, 1)
        return carry

    lax.fori_loop(0, NB // 2, outer, 0)

    store_copy(NB - 2, 0).wait()
    store_copy(NB - 1, 1).wait()


def kernel(x, table):
    assert x.shape == (NROW, NCOL) and table.shape[1] == D
    x5 = x.T.reshape(NCOL, TI, 128)
    out5 = _sc_embedding(x5, table)
    return out5.transpose(2, 4, 0, 1, 3).reshape(NROW, NCOL, D)
